# baseline (speedup 1.0000x reference)
import functools

import jax
import jax.numpy as jnp
from jax import lax
from jax.experimental import pallas as pl
from jax.experimental.pallas import tpu as pltpu
from jax.experimental.pallas import tpu_sc as plsc

IDX_CHUNK = 128


@functools.lru_cache(maxsize=None)
def _make_gather(B: int, E: int):
    info = plsc.get_sparse_core_info()
    nc, ns = info.num_cores, info.num_subcores
    nw = nc * ns
    b_per_w = B // nw
    assert B % (8 * nw) == 0
    n_chunks = b_per_w // IDX_CHUNK
    assert b_per_w % IDX_CHUNK == 0

    mesh = plsc.VectorSubcoreMesh(core_axis_name="c", subcore_axis_name="s")

    @functools.partial(
        pl.kernel,
        mesh=mesh,
        out_type=jax.ShapeDtypeStruct((4, B, E), jnp.float32),
        scratch_types=[
            pltpu.VMEM((b_per_w,), jnp.int32),
            pltpu.VMEM((b_per_w, E), jnp.float32),
            pltpu.SemaphoreType.DMA,
        ],
        compiler_params=pltpu.CompilerParams(use_tc_tiling_on_sc=False),
    )
    def gather_k(oi, pi, ri, vi, ot, pt, rt, vt, out, idx_v, rows_v, sem):
        wid = lax.axis_index("s") * nc + lax.axis_index("c")
        base = wid * b_per_w
        for t, (ih, th) in enumerate(((oi, ot), (pi, pt), (ri, rt), (vi, vt))):
            pltpu.sync_copy(ih.at[pl.ds(base, b_per_w)], idx_v)
            cps = []
            for j in range(n_chunks):
                cps.append(
                    pltpu.async_copy(
                        th.at[idx_v.at[pl.ds(j * IDX_CHUNK, IDX_CHUNK)]],
                        rows_v.at[pl.ds(j * IDX_CHUNK, IDX_CHUNK)],
                        sem,
                    )
                )
            for cp in cps:
                cp.wait()
            pltpu.sync_copy(rows_v, out.at[t].at[pl.ds(base, b_per_w)])

    return gather_k


def _format_body(x_ref, o_ref):
    x = x_ref[...]
    cw = x.shape[1]
    xT = jnp.transpose(x)
    xr = jnp.reshape(xT, (cw // 4, 4, x.shape[0]))
    o_ref[...] = jnp.concatenate([xr[:, u, :] for u in range(4)], axis=1)


@functools.lru_cache(maxsize=None)
def _make_format(N: int, E: int, CW: int):
    return pl.pallas_call(
        _format_body,
        grid=(pl.cdiv(N, CW),),
        in_specs=[pl.BlockSpec((E, CW), lambda i: (0, i))],
        out_specs=pl.BlockSpec((CW // 4, 4 * E), lambda i: (i, 0)),
        out_shape=jax.ShapeDtypeStruct((N // 4, 4 * E), jnp.float32),
    )


def _make_assemble_body(BM, E):
    def body(g_ref, fa_ref, cf_ref, fw_ref, fb_ref, cw_ref, cb_ref, out_ref):
        gs = [g_ref[t] for t in range(4)]
        rows = []
        for r in range(4):
            lo, hi = E * r, E * r + E
            rows.append(jnp.concatenate([gs[t][:, lo:hi] for t in range(4)], axis=1))
        gblk = jnp.stack(rows, axis=1).reshape(BM, 4 * E)
        gT = jnp.transpose(gblk)
        flT = (
            lax.dot_general(
                fw_ref[...],
                fa_ref[...],
                (((0,), (1,)), ((), ())),
                preferred_element_type=jnp.float32,
            )
            + fb_ref[...]
        )
        ctT = (
            lax.dot_general(
                cw_ref[...],
                cf_ref[...],
                (((0,), (0,)), ((), ())),
                preferred_element_type=jnp.float32,
            )
            + cb_ref[...]
        )
        out_ref[...] = jnp.concatenate([gT, flT, ctT], axis=0)

    return body


@functools.lru_cache(maxsize=None)
def _make_assemble(B: int, E: int, F: int, BM: int):
    BMq = BM // 4
    return pl.pallas_call(
        _make_assemble_body(BM, E),
        grid=(B // BM,),
        in_specs=[
            pl.BlockSpec((4, BMq, 4 * E), lambda i: (0, i, 0)),
            pl.BlockSpec((BM, F), lambda i: (i, 0)),
            pl.BlockSpec((3, BM), lambda i: (0, i)),
            pl.BlockSpec((F, E), lambda i: (0, 0)),
            pl.BlockSpec((E, 1), lambda i: (0, 0)),
            pl.BlockSpec((3, E), lambda i: (0, 0)),
            pl.BlockSpec((E, 1), lambda i: (0, 0)),
        ],
        out_specs=pl.BlockSpec((6 * E, BM), lambda i: (0, i)),
        out_shape=jax.ShapeDtypeStruct((6 * E, B), jnp.float32),
    )


def kernel(
    origin,
    process,
    roast_level,
    variety,
    flavors,
    target_finish_temp,
    altitude,
    bean_density,
    origin_table,
    process_table,
    roast_table,
    variety_table,
    flavor_W,
    flavor_b,
    cont_W,
    cont_b,
):
    B, F = flavors.shape
    E = origin_table.shape[1]
    oi = origin.reshape(B).astype(jnp.int32)
    pi = process.reshape(B).astype(jnp.int32)
    ri = roast_level.reshape(B).astype(jnp.int32)
    vi = variety.reshape(B).astype(jnp.int32)

    def _row_major(t):
        n = t.shape[0]
        packed = _make_format(n, E, 2560)(t.T)
        return jnp.reshape(packed, t.shape)

    g = _make_gather(B, E)(
        oi,
        pi,
        ri,
        vi,
        _row_major(origin_table),
        process_table,
        roast_table,
        _row_major(variety_table),
    )
    g = jnp.reshape(g, (4, B // 4, 4 * E))
    cfT = jnp.concatenate(
        [target_finish_temp.T, altitude.T, bean_density.T], axis=0
    )
    outT = _make_assemble(B, E, F, 1024)(
        g,
        flavors,
        cfT,
        flavor_W,
        flavor_b.reshape(E, 1),
        cont_W,
        cont_b.reshape(E, 1),
    )
    return outT.T
